# baseline (speedup 1.0000x reference)
import functools

import jax
import jax.numpy as jnp
from jax import lax
from jax.experimental import pallas as pl
from jax.experimental.pallas import tpu as pltpu
from jax.experimental.pallas import tpu_sc as plsc


def _scores_body(x_ref, w_ref, bias_ref, scores_ref, g_ref):
    xu = lax.bitcast_convert_type(x_ref[...], jnp.uint32)
    xu = (xu + jnp.uint32(0x7FFF) + ((xu >> jnp.uint32(16)) & jnp.uint32(1))
          ) & jnp.uint32(0xFFFF0000)
    xb = lax.bitcast_convert_type(xu, jnp.float32)
    wv = w_ref[...].astype(jnp.bfloat16).astype(jnp.float32)
    s = jnp.sum(xb * wv, axis=-1)
    s = s + bias_ref[0, 0]
    scores_ref[0, 0, :] = s
    g_ref[0, 0, :] = jax.nn.sigmoid(s)


def _compute_scores(x2, W, bias):
    n, h = x2.shape
    rows = 2048
    grid = n // rows
    return pl.pallas_call(
        _scores_body,
        grid=(grid,),
        in_specs=[
            pl.BlockSpec((rows, h), lambda i: (i, 0)),
            pl.BlockSpec((1, h), lambda i: (0, 0)),
            pl.BlockSpec(memory_space=pltpu.SMEM),
        ],
        out_specs=[
            pl.BlockSpec((1, 1, rows), lambda i: (i, 0, 0)),
            pl.BlockSpec((1, 1, rows), lambda i: (i, 0, 0)),
        ],
        out_shape=[
            jax.ShapeDtypeStruct((grid, 1, rows), jnp.float32),
            jax.ShapeDtypeStruct((grid, 1, rows), jnp.float32),
        ],
        compiler_params=pltpu.CompilerParams(
            dimension_semantics=("arbitrary",),
        ),
    )(x2, W, bias)


_LANES = 16
_NT = 16
_NW = 32


def _keys16(v):
    uu = lax.bitcast_convert_type(v, jnp.uint32)
    neg = uu >= jnp.uint32(0x80000000)
    xm = jnp.where(neg, jnp.uint32(0xFFFFFFFF), jnp.uint32(0x80000000))
    return uu ^ xm


def _make_select(n, k):
    cnt = n // _NT
    out_chunk = n // _NW
    cvecs = cnt // _LANES
    ovecs = out_chunk // _LANES

    mesh = plsc.VectorSubcoreMesh(
        core_axis_name="c", subcore_axis_name="s", num_cores=1)

    @functools.partial(
        pl.kernel,
        out_type=jax.ShapeDtypeStruct((n,), jnp.float32),
        mesh=mesh,
        compiler_params=pltpu.CompilerParams(needs_layout_passes=False),
        scratch_types=[
            pltpu.VMEM((cnt,), jnp.float32),
            pltpu.VMEM((cnt,), jnp.uint32),
            pltpu.VMEM((_LANES,), jnp.int32),
            pltpu.VMEM((_NT, _LANES), jnp.int32),
            pltpu.VMEM_SHARED((3, _NT, _LANES), jnp.int32),
            pltpu.VMEM((cnt,), jnp.float32),
        ],
    )
    def select(scores_hbm, out_hbm, sc_v, keys_v, stage_v, read_v, shared,
               outm_v):
        t = lax.axis_index("s")
        lane = lax.broadcasted_iota(jnp.int32, (_LANES,), 0)

        pltpu.sync_copy(scores_hbm.at[pl.ds(t * cnt, cnt)], sc_v)

        def _kb(i, _):
            for u in range(8):
                off = (i * 8 + u) * _LANES
                keys_v[pl.ds(off, _LANES)] = _keys16(sc_v[pl.ds(off, _LANES)])
            return 0
        lax.fori_loop(0, cvecs // 8, _kb, 0)

        prefix = jnp.uint32(0)
        for r in range(8):
            shift = 28 - 4 * r
            cands = [prefix | jnp.uint32(j << shift) for j in range(1, 16)]

            def _cb(i, accs):
                new = list(accs)
                for u in range(4):
                    off = (i * 4 + u) * _LANES
                    kv = keys_v[pl.ds(off, _LANES)]
                    for j in range(15):
                        new[j] = new[j] + jnp.where(kv >= cands[j], 1, 0)
                return tuple(new)

            accs = lax.fori_loop(
                0, cvecs // 4, _cb,
                tuple(jnp.zeros((_LANES,), jnp.int32) for _ in range(15)))

            counts = jnp.full((_LANES,), n, jnp.int32)
            for j in range(15):
                counts = jnp.where(lane == j + 1, jnp.sum(accs[j]), counts)

            parity = r % 2
            stage_v[...] = counts
            pltpu.sync_copy(stage_v, shared.at[parity, t])
            plsc.subcore_barrier()
            pltpu.sync_copy(shared.at[parity], read_v)
            tot = jnp.zeros((_LANES,), jnp.int32)
            for tt in range(_NT):
                tot = tot + read_v[tt]
            cond = (tot >= k).astype(jnp.int32)
            jstar = jnp.sum(cond) - 1
            prefix = prefix | (jstar.astype(jnp.uint32)
                               << jnp.uint32(shift))

        thr = prefix

        def _eb(i, carry):
            gt, eq = carry
            for u in range(8):
                off = (i * 8 + u) * _LANES
                kv = keys_v[pl.ds(off, _LANES)]
                gt = gt + jnp.where(kv > thr, 1, 0)
                eq = eq + jnp.where(kv == thr, 1, 0)
            return gt, eq

        z = jnp.zeros((_LANES,), jnp.int32)
        gt, eq = lax.fori_loop(0, cvecs // 8, _eb, (z, z))

        info = jnp.where(lane == 0, jnp.sum(gt), jnp.sum(eq))
        stage_v[...] = info
        pltpu.sync_copy(stage_v, shared.at[2, t])
        plsc.subcore_barrier()
        pltpu.sync_copy(shared.at[2], read_v)

        gt_total = jnp.int32(0)
        eq_before = jnp.int32(0)
        for tt in range(_NT):
            row = read_v[tt]
            gt_total = gt_total + jnp.sum(jnp.where(lane == 0, row, 0))
            eq_t = jnp.sum(jnp.where(lane == 1, row, 0))
            eq_before = eq_before + jnp.where(tt < t, eq_t, 0)
        needed = k - gt_total

        def _mb(i, carry):
            for u in range(8):
                off = (i * 8 + u) * _LANES
                ku = keys_v[pl.ds(off, _LANES)]
                eqm = ku == thr
                eqi = jnp.where(eqm, 1, 0)
                incl = plsc.cumsum(eqi)
                rank = carry + (incl - eqi)
                sel = (ku > thr) | (eqm & (rank < needed))
                outm_v[pl.ds(off, _LANES)] = jnp.where(sel, 1.0, 0.0)
                carry = carry + jnp.sum(eqi)
            return carry

        lax.fori_loop(0, cvecs // 8, _mb, eq_before)
        pltpu.sync_copy(outm_v, out_hbm.at[pl.ds(t * cnt, cnt)])

    return select


def kernel(x, W, b, step_embed, step):
    Bd, Sd, Hd = x.shape
    n = Bd * Sd
    k = max(1, int(round(n * 0.25)))

    x2 = x.reshape(n, Hd)
    bias = (b[0] + step_embed[step, 0]).reshape(1, 1).astype(jnp.float32)

    scores2, g2 = _compute_scores(x2, W.astype(jnp.float32), bias)
    scores_flat = scores2.reshape(n)

    m_flat = _make_select(n, k)(scores_flat)

    g = g2.reshape(Bd, Sd, 1)
    m = m_flat.reshape(Bd, Sd, 1)
    aux_loss = jnp.zeros((), x.dtype)
    return (g, m, aux_loss)


# speedup vs baseline: 1.0915x; 1.0915x over previous
import functools

import jax
import jax.numpy as jnp
from jax import lax
from jax.experimental import pallas as pl
from jax.experimental.pallas import tpu as pltpu
from jax.experimental.pallas import tpu_sc as plsc


def _scores_body(x_ref, w_ref, bias_ref, scores_ref, g_ref):
    s = lax.dot_general(
        x_ref[...], w_ref[...],
        (((1,), (1,)), ((), ())),
        preferred_element_type=jnp.float32,
    )[:, 0]
    s = s + bias_ref[0, 0]
    scores_ref[0, 0, :] = s
    g_ref[0, 0, :] = jax.nn.sigmoid(s)


def _compute_scores(x2, W, bias):
    n, h = x2.shape
    rows = 2048
    grid = n // rows
    return pl.pallas_call(
        _scores_body,
        grid=(grid,),
        in_specs=[
            pl.BlockSpec((rows, h), lambda i: (i, 0)),
            pl.BlockSpec((1, h), lambda i: (0, 0)),
            pl.BlockSpec(memory_space=pltpu.SMEM),
        ],
        out_specs=[
            pl.BlockSpec((1, 1, rows), lambda i: (i, 0, 0)),
            pl.BlockSpec((1, 1, rows), lambda i: (i, 0, 0)),
        ],
        out_shape=[
            jax.ShapeDtypeStruct((grid, 1, rows), jnp.float32),
            jax.ShapeDtypeStruct((grid, 1, rows), jnp.float32),
        ],
        compiler_params=pltpu.CompilerParams(
            dimension_semantics=("arbitrary",),
        ),
    )(x2, W, bias)


_LANES = 16
_NT = 16
_NW = 32


def _keys16(v):
    uu = lax.bitcast_convert_type(v, jnp.uint32)
    neg = uu >= jnp.uint32(0x80000000)
    xm = jnp.where(neg, jnp.uint32(0xFFFFFFFF), jnp.uint32(0x80000000))
    return uu ^ xm


def _make_select(n, k):
    cnt = n // _NT
    out_chunk = n // _NW
    cvecs = cnt // _LANES
    ovecs = out_chunk // _LANES

    mesh = plsc.VectorSubcoreMesh(
        core_axis_name="c", subcore_axis_name="s", num_cores=1)

    @functools.partial(
        pl.kernel,
        out_type=jax.ShapeDtypeStruct((n,), jnp.float32),
        mesh=mesh,
        compiler_params=pltpu.CompilerParams(needs_layout_passes=False),
        scratch_types=[
            pltpu.VMEM((cnt,), jnp.float32),
            pltpu.VMEM((cnt,), jnp.uint32),
            pltpu.VMEM((_LANES,), jnp.int32),
            pltpu.VMEM((_NT, _LANES), jnp.int32),
            pltpu.VMEM_SHARED((3, _NT, _LANES), jnp.int32),
            pltpu.VMEM((cnt,), jnp.float32),
        ],
    )
    def select(scores_hbm, out_hbm, sc_v, keys_v, stage_v, read_v, shared,
               outm_v):
        t = lax.axis_index("s")
        lane = lax.broadcasted_iota(jnp.int32, (_LANES,), 0)

        pltpu.sync_copy(scores_hbm.at[pl.ds(t * cnt, cnt)], sc_v)

        def _kb(i, _):
            for u in range(8):
                off = (i * 8 + u) * _LANES
                keys_v[pl.ds(off, _LANES)] = _keys16(sc_v[pl.ds(off, _LANES)])
            return 0
        lax.fori_loop(0, cvecs // 8, _kb, 0)

        prefix = jnp.uint32(0)
        for r in range(8):
            shift = 28 - 4 * r
            cands = [prefix | jnp.uint32(j << shift) for j in range(1, 16)]

            def _cb(i, accs):
                new = list(accs)
                for u in range(4):
                    off = (i * 4 + u) * _LANES
                    kv = keys_v[pl.ds(off, _LANES)]
                    for j in range(15):
                        new[j] = new[j] + jnp.where(kv >= cands[j], 1, 0)
                return tuple(new)

            accs = lax.fori_loop(
                0, cvecs // 4, _cb,
                tuple(jnp.zeros((_LANES,), jnp.int32) for _ in range(15)))

            counts = jnp.full((_LANES,), n, jnp.int32)
            for j in range(15):
                counts = jnp.where(lane == j + 1, jnp.sum(accs[j]), counts)

            parity = r % 2
            stage_v[...] = counts
            pltpu.sync_copy(stage_v, shared.at[parity, t])
            plsc.subcore_barrier()
            pltpu.sync_copy(shared.at[parity], read_v)
            tot = jnp.zeros((_LANES,), jnp.int32)
            for tt in range(_NT):
                tot = tot + read_v[tt]
            cond = (tot >= k).astype(jnp.int32)
            jstar = jnp.sum(cond) - 1
            prefix = prefix | (jstar.astype(jnp.uint32)
                               << jnp.uint32(shift))

        thr = prefix

        def _eb(i, carry):
            gt, eq = carry
            for u in range(8):
                off = (i * 8 + u) * _LANES
                kv = keys_v[pl.ds(off, _LANES)]
                gt = gt + jnp.where(kv > thr, 1, 0)
                eq = eq + jnp.where(kv == thr, 1, 0)
            return gt, eq

        z = jnp.zeros((_LANES,), jnp.int32)
        gt, eq = lax.fori_loop(0, cvecs // 8, _eb, (z, z))

        info = jnp.where(lane == 0, jnp.sum(gt), jnp.sum(eq))
        stage_v[...] = info
        pltpu.sync_copy(stage_v, shared.at[2, t])
        plsc.subcore_barrier()
        pltpu.sync_copy(shared.at[2], read_v)

        gt_total = jnp.int32(0)
        eq_before = jnp.int32(0)
        for tt in range(_NT):
            row = read_v[tt]
            gt_total = gt_total + jnp.sum(jnp.where(lane == 0, row, 0))
            eq_t = jnp.sum(jnp.where(lane == 1, row, 0))
            eq_before = eq_before + jnp.where(tt < t, eq_t, 0)
        needed = k - gt_total

        def _mb(i, carry):
            for u in range(8):
                off = (i * 8 + u) * _LANES
                ku = keys_v[pl.ds(off, _LANES)]
                eqm = ku == thr
                eqi = jnp.where(eqm, 1, 0)
                incl = plsc.cumsum(eqi)
                rank = carry + (incl - eqi)
                sel = (ku > thr) | (eqm & (rank < needed))
                outm_v[pl.ds(off, _LANES)] = jnp.where(sel, 1.0, 0.0)
                carry = carry + jnp.sum(eqi)
            return carry

        lax.fori_loop(0, cvecs // 8, _mb, eq_before)
        pltpu.sync_copy(outm_v, out_hbm.at[pl.ds(t * cnt, cnt)])

    return select


def kernel(x, W, b, step_embed, step):
    Bd, Sd, Hd = x.shape
    n = Bd * Sd
    k = max(1, int(round(n * 0.25)))

    x2 = x.reshape(n, Hd)
    bias = (b[0] + step_embed[step, 0]).reshape(1, 1).astype(jnp.float32)

    scores2, g2 = _compute_scores(x2, W.astype(jnp.float32), bias)
    scores_flat = scores2.reshape(n)

    m_flat = _make_select(n, k)(scores_flat)

    g = g2.reshape(Bd, Sd, 1)
    m = m_flat.reshape(Bd, Sd, 1)
    aux_loss = jnp.zeros((), x.dtype)
    return (g, m, aux_loss)


# speedup vs baseline: 1.2782x; 1.1710x over previous
import functools

import jax
import jax.numpy as jnp
from jax import lax
from jax.experimental import pallas as pl
from jax.experimental.pallas import tpu as pltpu
from jax.experimental.pallas import tpu_sc as plsc


def _scores_body(x_ref, w_ref, bias_ref, scores_ref, g_ref):
    xb = x_ref[...].astype(jnp.bfloat16).astype(jnp.float32)
    wv = w_ref[...].astype(jnp.bfloat16).astype(jnp.float32)
    s = jnp.sum(xb * wv, axis=-1)
    s = s + bias_ref[0, 0]
    scores_ref[0, 0, :] = s
    g_ref[0, 0, :] = jax.nn.sigmoid(s)


def _compute_scores(x2, W, bias):
    n, h = x2.shape
    rows = 2048
    grid = n // rows
    return pl.pallas_call(
        _scores_body,
        grid=(grid,),
        in_specs=[
            pl.BlockSpec((rows, h), lambda i: (i, 0)),
            pl.BlockSpec((1, h), lambda i: (0, 0)),
            pl.BlockSpec(memory_space=pltpu.SMEM),
        ],
        out_specs=[
            pl.BlockSpec((1, 1, rows), lambda i: (i, 0, 0)),
            pl.BlockSpec((1, 1, rows), lambda i: (i, 0, 0)),
        ],
        out_shape=[
            jax.ShapeDtypeStruct((grid, 1, rows), jnp.float32),
            jax.ShapeDtypeStruct((grid, 1, rows), jnp.float32),
        ],
        compiler_params=pltpu.CompilerParams(
            dimension_semantics=("arbitrary",),
        ),
    )(x2, W, bias)


_LANES = 16
_NT = 16
_NW = 32


def _keys16(v):
    uu = lax.bitcast_convert_type(v, jnp.uint32)
    neg = uu >= jnp.uint32(0x80000000)
    xm = jnp.where(neg, jnp.uint32(0xFFFFFFFF), jnp.uint32(0x80000000))
    return uu ^ xm


def _make_select(n, k):
    cnt = n // _NT
    out_chunk = n // _NW
    cvecs = cnt // _LANES
    ovecs = out_chunk // _LANES

    mesh = plsc.VectorSubcoreMesh(
        core_axis_name="c", subcore_axis_name="s", num_cores=1)

    @functools.partial(
        pl.kernel,
        out_type=jax.ShapeDtypeStruct((n,), jnp.float32),
        mesh=mesh,
        compiler_params=pltpu.CompilerParams(needs_layout_passes=False),
        scratch_types=[
            pltpu.VMEM((cnt,), jnp.float32),
            pltpu.VMEM((cnt,), jnp.uint32),
            pltpu.VMEM((256,), jnp.int32),
            pltpu.VMEM((256,), jnp.int32),
            pltpu.VMEM((_NT, 256), jnp.int32),
            pltpu.VMEM((_LANES,), jnp.int32),
            pltpu.VMEM((_NT, _LANES), jnp.int32),
            pltpu.VMEM_SHARED((2, _NT, 256), jnp.int32),
            pltpu.VMEM_SHARED((_NT, _LANES), jnp.int32),
            pltpu.VMEM((cnt,), jnp.float32),
        ],
    )
    def select(scores_hbm, out_hbm, sc_v, keys_v, hist_v, g_v, readh_v,
               stage_v, read_v, shared_h, shared_i, outm_v):
        t = lax.axis_index("s")
        lane = lax.broadcasted_iota(jnp.int32, (_LANES,), 0)
        zero16 = jnp.zeros((_LANES,), jnp.int32)

        pltpu.sync_copy(scores_hbm.at[pl.ds(t * cnt, cnt)], sc_v)

        def _kb(i, _):
            for u in range(8):
                off = (i * 8 + u) * _LANES
                keys_v[pl.ds(off, _LANES)] = _keys16(sc_v[pl.ds(off, _LANES)])
            return 0
        lax.fori_loop(0, cvecs // 8, _kb, 0)

        prefix = jnp.uint32(0)
        cglobal = jnp.int32(n)
        for r in range(4):
            shift = 24 - 8 * r
            for b in range(256 // _LANES):
                hist_v[pl.ds(b * _LANES, _LANES)] = zero16
                g_v[pl.ds(b * _LANES, _LANES)] = zero16

            def _hb(i, _):
                for u in range(8):
                    off = (i * 8 + u) * _LANES
                    kv = keys_v[pl.ds(off, _LANES)]
                    byte = ((kv >> jnp.uint32(shift))
                            & jnp.uint32(255)).astype(jnp.int32)
                    if r == 0:
                        dcnt, lastm = plsc.scan_count(byte)
                    else:
                        active = (kv >> jnp.uint32(shift + 8)) == (
                            prefix >> jnp.uint32(shift + 8))
                        dcnt, lastm = plsc.scan_count(byte, active)
                    plsc.addupdate_scatter(hist_v, [byte], dcnt, mask=lastm)
                return 0
            lax.fori_loop(0, cvecs // 8, _hb, 0)

            parity = r % 2
            pltpu.sync_copy(hist_v, shared_h.at[parity, t])
            plsc.subcore_barrier()
            pltpu.sync_copy(shared_h.at[parity], readh_v)

            for tt in range(_NT):
                for b in range(256 // _LANES):
                    plsc.addupdate(g_v.at[pl.ds(b * _LANES, _LANES)],
                                   readh_v[tt, pl.ds(b * _LANES, _LANES)])

            tot_all = jnp.int32(0)
            for b in range(256 // _LANES):
                tot_all = tot_all + jnp.sum(g_v[pl.ds(b * _LANES, _LANES)])
            above = cglobal - tot_all
            run = above
            ncond = jnp.int32(0)
            cmin = jnp.int32(n)
            for b in range(256 // _LANES - 1, -1, -1):
                gv = g_v[pl.ds(b * _LANES, _LANES)]
                incl = plsc.cumsum(gv)
                tot_v = jnp.sum(gv)
                s_vec = run + (tot_v - incl + gv)
                cond = s_vec >= k
                ncond = ncond + jnp.sum(cond.astype(jnp.int32))
                cmin = jnp.minimum(
                    cmin, jnp.min(jnp.where(cond, s_vec, n)))
                run = run + tot_v
            jstar = ncond - 1
            prefix = prefix | (jstar.astype(jnp.uint32)
                               << jnp.uint32(shift))
            cglobal = cmin

        thr = prefix

        def _eb(i, carry):
            gt, eq = carry
            for u in range(8):
                off = (i * 8 + u) * _LANES
                kv = keys_v[pl.ds(off, _LANES)]
                gt = gt + jnp.where(kv > thr, 1, 0)
                eq = eq + jnp.where(kv == thr, 1, 0)
            return gt, eq

        z = jnp.zeros((_LANES,), jnp.int32)
        gt, eq = lax.fori_loop(0, cvecs // 8, _eb, (z, z))

        info = jnp.where(lane == 0, jnp.sum(gt), jnp.sum(eq))
        stage_v[...] = info
        pltpu.sync_copy(stage_v, shared_i.at[t])
        plsc.subcore_barrier()
        pltpu.sync_copy(shared_i, read_v)

        gt_total = jnp.int32(0)
        eq_before = jnp.int32(0)
        for tt in range(_NT):
            row = read_v[tt]
            gt_total = gt_total + jnp.sum(jnp.where(lane == 0, row, 0))
            eq_t = jnp.sum(jnp.where(lane == 1, row, 0))
            eq_before = eq_before + jnp.where(tt < t, eq_t, 0)
        needed = k - gt_total

        def _mb(i, carry):
            for u in range(8):
                off = (i * 8 + u) * _LANES
                ku = keys_v[pl.ds(off, _LANES)]
                eqm = ku == thr
                eqi = jnp.where(eqm, 1, 0)
                incl = plsc.cumsum(eqi)
                rank = carry + (incl - eqi)
                sel = (ku > thr) | (eqm & (rank < needed))
                outm_v[pl.ds(off, _LANES)] = jnp.where(sel, 1.0, 0.0)
                carry = carry + jnp.sum(eqi)
            return carry

        lax.fori_loop(0, cvecs // 8, _mb, eq_before)
        pltpu.sync_copy(outm_v, out_hbm.at[pl.ds(t * cnt, cnt)])

    return select


def kernel(x, W, b, step_embed, step):
    Bd, Sd, Hd = x.shape
    n = Bd * Sd
    k = max(1, int(round(n * 0.25)))

    x2 = x.reshape(n, Hd)
    bias = (b[0] + step_embed[step, 0]).reshape(1, 1).astype(jnp.float32)

    scores2, g2 = _compute_scores(x2, W.astype(jnp.float32), bias)
    scores_flat = scores2.reshape(n)

    m_flat = _make_select(n, k)(scores_flat)

    g = g2.reshape(Bd, Sd, 1)
    m = m_flat.reshape(Bd, Sd, 1)
    aux_loss = jnp.zeros((), x.dtype)
    return (g, m, aux_loss)


# speedup vs baseline: 1.3230x; 1.0351x over previous
import functools

import jax
import jax.numpy as jnp
from jax import lax
from jax.experimental import pallas as pl
from jax.experimental.pallas import tpu as pltpu
from jax.experimental.pallas import tpu_sc as plsc


def _scores_body(x_ref, w_ref, bias_ref, scores_ref, g_ref):
    xb = x_ref[...].astype(jnp.bfloat16).astype(jnp.float32)
    wv = w_ref[...].astype(jnp.bfloat16).astype(jnp.float32)
    s = jnp.sum(xb * wv, axis=-1)
    s = s + bias_ref[0, 0]
    scores_ref[0, 0, :] = s
    g_ref[0, 0, :] = jax.nn.sigmoid(s)


def _compute_scores(x2, W, bias):
    n, h = x2.shape
    rows = 2048
    grid = n // rows
    return pl.pallas_call(
        _scores_body,
        grid=(grid,),
        in_specs=[
            pl.BlockSpec((rows, h), lambda i: (i, 0)),
            pl.BlockSpec((1, h), lambda i: (0, 0)),
            pl.BlockSpec(memory_space=pltpu.SMEM),
        ],
        out_specs=[
            pl.BlockSpec((1, 1, rows), lambda i: (i, 0, 0)),
            pl.BlockSpec((1, 1, rows), lambda i: (i, 0, 0)),
        ],
        out_shape=[
            jax.ShapeDtypeStruct((grid, 1, rows), jnp.float32),
            jax.ShapeDtypeStruct((grid, 1, rows), jnp.float32),
        ],
        compiler_params=pltpu.CompilerParams(
            dimension_semantics=("arbitrary",),
        ),
    )(x2, W, bias)


_LANES = 16
_NT = 16
_NW = 32


def _keys16(v):
    uu = lax.bitcast_convert_type(v, jnp.uint32)
    neg = uu >= jnp.uint32(0x80000000)
    xm = jnp.where(neg, jnp.uint32(0xFFFFFFFF), jnp.uint32(0x80000000))
    return uu ^ xm


def _make_select(n, k):
    cnt = n // _NT
    out_chunk = n // _NW
    cvecs = cnt // _LANES
    ovecs = out_chunk // _LANES

    mesh = plsc.VectorSubcoreMesh(
        core_axis_name="c", subcore_axis_name="s", num_cores=1)

    @functools.partial(
        pl.kernel,
        out_type=jax.ShapeDtypeStruct((n,), jnp.float32),
        mesh=mesh,
        compiler_params=pltpu.CompilerParams(needs_layout_passes=False),
        scratch_types=[
            pltpu.VMEM((cnt,), jnp.float32),
            pltpu.VMEM((cnt,), jnp.uint32),
            pltpu.VMEM((256,), jnp.int32),
            pltpu.VMEM((256,), jnp.int32),
            pltpu.VMEM((_NT, 256), jnp.int32),
            pltpu.VMEM_SHARED((2, _NT, 256), jnp.int32),
            pltpu.VMEM((cnt,), jnp.float32),
        ],
    )
    def select(scores_hbm, out_hbm, sc_v, keys_v, hist_v, g_v, readh_v,
               shared_h, outm_v):
        t = lax.axis_index("s")
        lane = lax.broadcasted_iota(jnp.int32, (_LANES,), 0)
        zero16 = jnp.zeros((_LANES,), jnp.int32)

        pltpu.sync_copy(scores_hbm.at[pl.ds(t * cnt, cnt)], sc_v)

        def _kb(i, _):
            for u in range(8):
                off = (i * 8 + u) * _LANES
                keys_v[pl.ds(off, _LANES)] = _keys16(sc_v[pl.ds(off, _LANES)])
            return 0
        lax.fori_loop(0, cvecs // 8, _kb, 0)

        prefix = jnp.uint32(0)
        cglobal = jnp.int32(n)
        for r in range(4):
            shift = 24 - 8 * r
            for b in range(256 // _LANES):
                hist_v[pl.ds(b * _LANES, _LANES)] = zero16

            def _hb(i, _):
                for u in range(8):
                    off = (i * 8 + u) * _LANES
                    kv = keys_v[pl.ds(off, _LANES)]
                    byte = ((kv >> jnp.uint32(shift))
                            & jnp.uint32(255)).astype(jnp.int32)
                    if r == 0:
                        dcnt, lastm = plsc.scan_count(byte)
                    else:
                        active = (kv >> jnp.uint32(shift + 8)) == (
                            prefix >> jnp.uint32(shift + 8))
                        dcnt, lastm = plsc.scan_count(byte, active)
                    plsc.addupdate_scatter(hist_v, [byte], dcnt, mask=lastm)
                return 0
            lax.fori_loop(0, cvecs // 8, _hb, 0)

            parity = r % 2
            pltpu.sync_copy(hist_v, shared_h.at[parity, t])
            plsc.subcore_barrier()
            pltpu.sync_copy(shared_h.at[parity], readh_v)

            for b in range(256 // _LANES):
                acc = readh_v[0, pl.ds(b * _LANES, _LANES)]
                for tt in range(1, _NT):
                    acc = acc + readh_v[tt, pl.ds(b * _LANES, _LANES)]
                g_v[pl.ds(b * _LANES, _LANES)] = acc

            tot_all = jnp.int32(0)
            for b in range(256 // _LANES):
                tot_all = tot_all + jnp.sum(g_v[pl.ds(b * _LANES, _LANES)])
            above = cglobal - tot_all
            run = above
            ncond = jnp.int32(0)
            cmin = jnp.int32(n)
            for b in range(256 // _LANES - 1, -1, -1):
                gv = g_v[pl.ds(b * _LANES, _LANES)]
                incl = plsc.cumsum(gv)
                tot_v = jnp.sum(gv)
                s_vec = run + (tot_v - incl + gv)
                cond = s_vec >= k
                ncond = ncond + jnp.sum(cond.astype(jnp.int32))
                cmin = jnp.minimum(
                    cmin, jnp.min(jnp.where(cond, s_vec, n)))
                run = run + tot_v
            jstar = ncond - 1
            prefix = prefix | (jstar.astype(jnp.uint32)
                               << jnp.uint32(shift))
            cglobal = cmin

        thr = prefix

        eq_total = jnp.int32(0)
        eq_before = jnp.int32(0)
        for b in range(256 // _LANES):
            sel_lane = lane == (jstar - b * _LANES)
            gv = g_v[pl.ds(b * _LANES, _LANES)]
            eq_total = eq_total + jnp.sum(jnp.where(sel_lane, gv, 0))
            pre = jnp.zeros((_LANES,), jnp.int32)
            for tt in range(_NT):
                row = readh_v[tt, pl.ds(b * _LANES, _LANES)]
                pre = pre + jnp.where(tt < t, row, 0)
            eq_before = eq_before + jnp.sum(jnp.where(sel_lane, pre, 0))
        gt_total = cglobal - eq_total
        needed = k - gt_total

        def _mb(i, carry):
            for u in range(8):
                off = (i * 8 + u) * _LANES
                ku = keys_v[pl.ds(off, _LANES)]
                eqm = ku == thr
                eqi = jnp.where(eqm, 1, 0)
                incl = plsc.cumsum(eqi)
                rank = carry + (incl - eqi)
                sel = (ku > thr) | (eqm & (rank < needed))
                outm_v[pl.ds(off, _LANES)] = jnp.where(sel, 1.0, 0.0)
                carry = carry + jnp.sum(eqi)
            return carry

        lax.fori_loop(0, cvecs // 8, _mb, eq_before)
        pltpu.sync_copy(outm_v, out_hbm.at[pl.ds(t * cnt, cnt)])

    return select


def kernel(x, W, b, step_embed, step):
    Bd, Sd, Hd = x.shape
    n = Bd * Sd
    k = max(1, int(round(n * 0.25)))

    x2 = x.reshape(n, Hd)
    bias = (b[0] + step_embed[step, 0]).reshape(1, 1).astype(jnp.float32)

    scores2, g2 = _compute_scores(x2, W.astype(jnp.float32), bias)
    scores_flat = scores2.reshape(n)

    m_flat = _make_select(n, k)(scores_flat)

    g = g2.reshape(Bd, Sd, 1)
    m = m_flat.reshape(Bd, Sd, 1)
    aux_loss = jnp.zeros((), x.dtype)
    return (g, m, aux_loss)
